# baseline (speedup 1.0000x reference)
import functools

import jax
import jax.numpy as jnp
from jax import lax
from jax.experimental import pallas as pl
from jax.experimental.pallas import tpu as pltpu
from jax.experimental.pallas import tpu_sc as plsc

_CHUNK = 128


@functools.lru_cache(maxsize=None)
def _make_gather(V, D, B):
    info = plsc.get_sparse_core_info()
    NC, NS = info.num_cores, info.num_subcores
    NW = NC * NS
    assert B % NW == 0
    b_per_w = B // NW
    assert b_per_w % _CHUNK == 0
    n_ch = b_per_w // _CHUNK

    mesh = plsc.VectorSubcoreMesh(core_axis_name="c", subcore_axis_name="s")

    @functools.partial(
        pl.kernel,
        mesh=mesh,
        compiler_params=pltpu.CompilerParams(use_tc_tiling_on_sc=False),
        out_type=jax.ShapeDtypeStruct((B, D), jnp.float32),
        scratch_types=[
            pltpu.VMEM((_CHUNK,), jnp.int32),
            pltpu.VMEM((_CHUNK, D), jnp.float32),
            pltpu.SemaphoreType.DMA,
        ],
    )
    def gather_kernel(table_hbm, idx_hbm, out_hbm, idx_v, rows_v, sem):
        wid = lax.axis_index("s") * NC + lax.axis_index("c")
        base = wid * b_per_w

        def body(g, carry):
            off = base + g * _CHUNK
            pltpu.sync_copy(idx_hbm.at[pl.ds(off, _CHUNK)], idx_v)
            pltpu.async_copy(table_hbm.at[idx_v], rows_v, sem).wait()
            pltpu.sync_copy(rows_v, out_hbm.at[pl.ds(off, _CHUNK)])
            return carry

        lax.fori_loop(0, n_ch, body, 0)

    return gather_kernel


def kernel(x, weight):
    Bt, H = x.shape
    V, D = weight.shape
    B = Bt * H
    xf = x.reshape(B).astype(jnp.int32)
    out = _make_gather(V, D, B)(weight, xf)
    return out.reshape(Bt, H, D)


# speedup vs baseline: 1.1918x; 1.1918x over previous
import functools

import jax
import jax.numpy as jnp
from jax import lax
from jax.experimental import pallas as pl
from jax.experimental.pallas import tpu as pltpu
from jax.experimental.pallas import tpu_sc as plsc

_CHUNK = 128
_NBUF = 4


@functools.lru_cache(maxsize=None)
def _make_gather(V, D, B):
    info = plsc.get_sparse_core_info()
    NC, NS = info.num_cores, info.num_subcores
    NW = NC * NS
    assert B % (NW * _CHUNK) == 0
    b_per_w = B // NW
    n_ch = b_per_w // _CHUNK
    assert n_ch % _NBUF == 0
    n_groups = n_ch // _NBUF

    mesh = plsc.VectorSubcoreMesh(core_axis_name="c", subcore_axis_name="s")

    rows_scratch = [pltpu.VMEM((_CHUNK, D), jnp.float32) for _ in range(_NBUF)]
    sem_scratch = [pltpu.SemaphoreType.DMA for _ in range(2 * _NBUF)]

    @functools.partial(
        pl.kernel,
        mesh=mesh,
        compiler_params=pltpu.CompilerParams(use_tc_tiling_on_sc=False),
        out_type=jax.ShapeDtypeStruct((B, D), jnp.float32),
        scratch_types=[pltpu.VMEM((n_ch, _CHUNK), jnp.int32)]
        + rows_scratch
        + sem_scratch,
    )
    def gather_kernel(table_hbm, idx_hbm, out_hbm, idx_slab, *bufs):
        rows = bufs[:_NBUF]
        sem_g = bufs[_NBUF : 2 * _NBUF]
        sem_s = bufs[2 * _NBUF :]
        wid = lax.axis_index("s") * NC + lax.axis_index("c")
        base = wid * b_per_w

        pltpu.sync_copy(idx_hbm.at[pl.ds(wid * n_ch, n_ch)], idx_slab)

        def gather_start(g, b):
            pltpu.async_copy(table_hbm.at[idx_slab.at[g]], rows[b], sem_g[b])

        def gather_wait(g, b):
            pltpu.make_async_copy(
                table_hbm.at[idx_slab.at[g]], rows[b], sem_g[b]
            ).wait()

        def store_start(g, b):
            pltpu.async_copy(
                rows[b], out_hbm.at[pl.ds(base + g * _CHUNK, _CHUNK)], sem_s[b]
            )

        def store_wait(g, b):
            pltpu.make_async_copy(
                rows[b], out_hbm.at[pl.ds(base + g * _CHUNK, _CHUNK)], sem_s[b]
            ).wait()

        for b in range(_NBUF):
            gather_start(b, b)

        def group_body(gg, carry):
            g0 = gg * _NBUF
            for b in range(_NBUF):
                gather_wait(g0 + b, b)
                store_start(g0 + b, b)
            for b in range(_NBUF):
                store_wait(g0 + b, b)
                gather_start(g0 + _NBUF + b, b)
            return carry

        lax.fori_loop(0, n_groups - 1, group_body, 0)

        g0 = (n_groups - 1) * _NBUF
        for b in range(_NBUF):
            gather_wait(g0 + b, b)
            store_start(g0 + b, b)
        for b in range(_NBUF):
            store_wait(g0 + b, b)

    return gather_kernel


def kernel(x, weight):
    Bt, H = x.shape
    V, D = weight.shape
    B = Bt * H
    xf = x.reshape(B // _CHUNK, _CHUNK).astype(jnp.int32)
    out = _make_gather(V, D, B)(weight, xf)
    return out.reshape(Bt, H, D)


# speedup vs baseline: 1.1954x; 1.0030x over previous
import functools

import jax
import jax.numpy as jnp
from jax import lax
from jax.experimental import pallas as pl
from jax.experimental.pallas import tpu as pltpu
from jax.experimental.pallas import tpu_sc as plsc

_CHUNK = 128
_NBUF = 8


@functools.lru_cache(maxsize=None)
def _make_gather(V, D, B):
    info = plsc.get_sparse_core_info()
    NC, NS = info.num_cores, info.num_subcores
    NW = NC * NS
    assert B % (NW * _CHUNK) == 0
    b_per_w = B // NW
    n_ch = b_per_w // _CHUNK
    assert n_ch % _NBUF == 0
    n_groups = n_ch // _NBUF

    mesh = plsc.VectorSubcoreMesh(core_axis_name="c", subcore_axis_name="s")

    rows_scratch = [pltpu.VMEM((_CHUNK, D), jnp.float32) for _ in range(_NBUF)]
    sem_scratch = [pltpu.SemaphoreType.DMA for _ in range(2 * _NBUF)]

    @functools.partial(
        pl.kernel,
        mesh=mesh,
        compiler_params=pltpu.CompilerParams(use_tc_tiling_on_sc=False),
        out_type=jax.ShapeDtypeStruct((B, D), jnp.float32),
        scratch_types=[pltpu.VMEM((n_ch, _CHUNK), jnp.int32)]
        + rows_scratch
        + sem_scratch,
    )
    def gather_kernel(table_hbm, idx_hbm, out_hbm, idx_slab, *bufs):
        rows = bufs[:_NBUF]
        sem_g = bufs[_NBUF : 2 * _NBUF]
        sem_s = bufs[2 * _NBUF :]
        wid = lax.axis_index("s") * NC + lax.axis_index("c")
        base = wid * b_per_w

        pltpu.sync_copy(idx_hbm.at[pl.ds(wid * n_ch, n_ch)], idx_slab)

        def gather_start(g, b):
            pltpu.async_copy(table_hbm.at[idx_slab.at[g]], rows[b], sem_g[b])

        def gather_wait(g, b):
            pltpu.make_async_copy(
                table_hbm.at[idx_slab.at[g]], rows[b], sem_g[b]
            ).wait()

        def store_start(g, b):
            pltpu.async_copy(
                rows[b], out_hbm.at[pl.ds(base + g * _CHUNK, _CHUNK)], sem_s[b]
            )

        def store_wait(g, b):
            pltpu.make_async_copy(
                rows[b], out_hbm.at[pl.ds(base + g * _CHUNK, _CHUNK)], sem_s[b]
            ).wait()

        for b in range(_NBUF):
            gather_start(b, b)

        def group_body(gg, carry):
            g0 = gg * _NBUF
            for b in range(_NBUF):
                gather_wait(g0 + b, b)
                store_start(g0 + b, b)
            for b in range(_NBUF):
                store_wait(g0 + b, b)
                gather_start(g0 + _NBUF + b, b)
            return carry

        lax.fori_loop(0, n_groups - 1, group_body, 0)

        g0 = (n_groups - 1) * _NBUF
        for b in range(_NBUF):
            gather_wait(g0 + b, b)
            store_start(g0 + b, b)
        for b in range(_NBUF):
            store_wait(g0 + b, b)

    return gather_kernel


def kernel(x, weight):
    Bt, H = x.shape
    V, D = weight.shape
    B = Bt * H
    xf = x.reshape(B // _CHUNK, _CHUNK).astype(jnp.int32)
    out = _make_gather(V, D, B)(weight, xf)
    return out.reshape(Bt, H, D)
